# Initial kernel scaffold; baseline (speedup 1.0000x reference)
import jax
import jax.numpy as jnp
from jax.experimental import pallas as pl


def kernel(atom_feats, bond_feats, global_feats, edge_index, atom2graph, bond2graph, norm_atom, norm_bond, emb_atom_w, emb_bond_w, emb_global_w, L0_A_w, L0_A_b, L0_B_w, L0_B_b, L0_C_w, L0_C_b, L0_D_w, L0_D_b, L0_E_w, L0_E_b, L0_F_w, L0_F_b, L0_G_w, L0_H_w, L0_bn_h_g, L0_bn_h_b, L0_bn_e_g, L0_bn_e_b, L0_bn_u_g, L0_bn_u_b, L1_A_w, L1_A_b, L1_B_w, L1_B_b, L1_C_w, L1_C_b, L1_D_w, L1_D_b, L1_E_w, L1_E_b, L1_F_w, L1_F_b, L1_G_w, L1_H_w, L1_bn_h_g, L1_bn_h_b, L1_bn_e_g, L1_bn_e_b, L1_bn_u_g, L1_bn_u_b, L2_A_w, L2_A_b, L2_B_w, L2_B_b, L2_C_w, L2_C_b, L2_D_w, L2_D_b, L2_E_w, L2_E_b, L2_F_w, L2_F_b, L2_G_w, L2_H_w, L2_bn_h_g, L2_bn_h_b, L2_bn_e_g, L2_bn_e_b, L2_bn_u_g, L2_bn_u_b, s2s_atom_Wih0, s2s_atom_Whh0, s2s_atom_bias0, s2s_atom_Wih1, s2s_atom_Whh1, s2s_atom_bias1, s2s_atom_Wih2, s2s_atom_Whh2, s2s_atom_bias2, s2s_bond_Wih0, s2s_bond_Whh0, s2s_bond_bias0, s2s_bond_Wih1, s2s_bond_Whh1, s2s_bond_bias1, s2s_bond_Wih2, s2s_bond_Whh2, s2s_bond_bias2, fc0_w, fc0_b, fc1_w, fc1_b, fc2_w, fc2_b):
    raise NotImplementedError("write your pallas kernel here")


# speedup vs baseline: 3.3704x; 3.3704x over previous
import functools

import jax
import jax.numpy as jnp
from jax import lax
from jax.experimental import pallas as pl
from jax.experimental.pallas import tpu as pltpu
from jax.experimental.pallas import tpu_sc as plsc

NA = 50000
NB = 800000
NG = 2000
BR = 2048
BP = 802816
AP = 53248
GP = 2048
NC = 2
NS = 16
CH = 128


def _mm(x, w, b=None):
    n, din = x.shape
    dout = w.shape[1]
    bias = jnp.zeros((1, dout), jnp.float32) if b is None else b.reshape(1, dout)

    def body(x_ref, w_ref, b_ref, o_ref):
        o_ref[...] = (
            jnp.dot(x_ref[...], w_ref[...], preferred_element_type=jnp.float32)
            + b_ref[...]
        )

    return pl.pallas_call(
        body,
        grid=(n // BR,),
        in_specs=[
            pl.BlockSpec((BR, din), lambda i: (i, 0)),
            pl.BlockSpec((din, dout), lambda i: (0, 0)),
            pl.BlockSpec((1, dout), lambda i: (0, 0)),
        ],
        out_specs=pl.BlockSpec((BR, dout), lambda i: (i, 0)),
        out_shape=jax.ShapeDtypeStruct((n, dout), jnp.float32),
    )(x, w, bias)


def _edge_pre(e, gs, gd, eu, nb_, w, bias, n_true):
    n, din = e.shape
    dout = w.shape[1]

    def body(e_ref, gs_ref, gd_ref, eu_ref, nb_ref, w_ref, b_ref, o_ref, p_ref):
        i = pl.program_id(0)
        t = (
            jnp.dot(e_ref[...], w_ref[...], preferred_element_type=jnp.float32)
            + b_ref[...]
        )
        t = t + gs_ref[:, dout:] + gd_ref[:, dout:] + eu_ref[...]
        t = t * nb_ref[...]
        rows = lax.broadcasted_iota(jnp.int32, (BR, 1), 0) + i * BR
        t = jnp.where(rows < n_true, t, 0.0)
        o_ref[...] = t

        @pl.when(i == 0)
        def _():
            p_ref[...] = jnp.zeros_like(p_ref)

        p_ref[0:1, :] += jnp.sum(t, axis=0, keepdims=True)
        p_ref[1:2, :] += jnp.sum(t * t, axis=0, keepdims=True)

    return pl.pallas_call(
        body,
        grid=(n // BR,),
        in_specs=[
            pl.BlockSpec((BR, din), lambda i: (i, 0)),
            pl.BlockSpec((BR, 2 * dout), lambda i: (i, 0)),
            pl.BlockSpec((BR, 2 * dout), lambda i: (i, 0)),
            pl.BlockSpec((BR, dout), lambda i: (i, 0)),
            pl.BlockSpec((BR, 1), lambda i: (i, 0)),
            pl.BlockSpec((din, dout), lambda i: (0, 0)),
            pl.BlockSpec((1, dout), lambda i: (0, 0)),
        ],
        out_specs=[
            pl.BlockSpec((BR, dout), lambda i: (i, 0)),
            pl.BlockSpec((8, dout), lambda i: (0, 0)),
        ],
        out_shape=[
            jax.ShapeDtypeStruct((n, dout), jnp.float32),
            jax.ShapeDtypeStruct((8, dout), jnp.float32),
        ],
    )(e, gs, gd, eu, nb_, w, bias)


def _edge_post(epre, gs, gd, st, e_res, n_true):
    n, dout = epre.shape
    resid = e_res is not None

    def body(*refs):
        if resid:
            ep_ref, gs_ref, gd_ref, st_ref, er_ref = refs[:5]
            outs = refs[5:]
        else:
            ep_ref, gs_ref, gd_ref, st_ref = refs[:4]
            outs = refs[4:]
        en_ref, sg_ref, m1_ref, m2_ref = outs
        i = pl.program_id(0)
        x = ep_ref[...] * st_ref[0:1, :] + st_ref[1:2, :]
        x = jnp.maximum(x, 0.0)
        if resid:
            x = x + er_ref[...]
        rows = lax.broadcasted_iota(jnp.int32, (BR, 1), 0) + i * BR
        mask = rows < n_true
        x = jnp.where(mask, x, 0.0)
        sg = jnp.where(mask, jax.nn.sigmoid(x), 0.0)
        en_ref[...] = x
        sg_ref[...] = sg
        m1_ref[...] = sg * gd_ref[:, :dout]
        m2_ref[...] = sg * gs_ref[:, :dout]

    in_specs = [
        pl.BlockSpec((BR, dout), lambda i: (i, 0)),
        pl.BlockSpec((BR, 2 * dout), lambda i: (i, 0)),
        pl.BlockSpec((BR, 2 * dout), lambda i: (i, 0)),
        pl.BlockSpec((8, dout), lambda i: (0, 0)),
    ]
    args = [epre, gs, gd, st]
    if resid:
        in_specs.append(pl.BlockSpec((BR, dout), lambda i: (i, 0)))
        args.append(e_res)
    sh = jax.ShapeDtypeStruct((n, dout), jnp.float32)
    return pl.pallas_call(
        body,
        grid=(n // BR,),
        in_specs=in_specs,
        out_specs=[pl.BlockSpec((BR, dout), lambda i: (i, 0))] * 4,
        out_shape=[sh, sh, sh, sh],
    )(*args)


def _atom_pre(ah, num, den, na_, n_true):
    n, dout = ah.shape

    def body(a_ref, nu_ref, de_ref, na_ref, o_ref, p_ref):
        i = pl.program_id(0)
        t = a_ref[...] + nu_ref[...] / (de_ref[...] + 1e-6)
        t = t * na_ref[...]
        rows = lax.broadcasted_iota(jnp.int32, (BR, 1), 0) + i * BR
        t = jnp.where(rows < n_true, t, 0.0)
        o_ref[...] = t

        @pl.when(i == 0)
        def _():
            p_ref[...] = jnp.zeros_like(p_ref)

        p_ref[0:1, :] += jnp.sum(t, axis=0, keepdims=True)
        p_ref[1:2, :] += jnp.sum(t * t, axis=0, keepdims=True)

    return pl.pallas_call(
        body,
        grid=(n // BR,),
        in_specs=[
            pl.BlockSpec((BR, dout), lambda i: (i, 0)),
            pl.BlockSpec((BR, dout), lambda i: (i, 0)),
            pl.BlockSpec((BR, dout), lambda i: (i, 0)),
            pl.BlockSpec((BR, 1), lambda i: (i, 0)),
        ],
        out_specs=[
            pl.BlockSpec((BR, dout), lambda i: (i, 0)),
            pl.BlockSpec((8, dout), lambda i: (0, 0)),
        ],
        out_shape=[
            jax.ShapeDtypeStruct((n, dout), jnp.float32),
            jax.ShapeDtypeStruct((8, dout), jnp.float32),
        ],
    )(ah, num, den, na_)


def _post_bn_relu(x, st, res, n_true):
    n, dout = x.shape
    resid = res is not None

    def body(*refs):
        if resid:
            x_ref, st_ref, r_ref, o_ref = refs
        else:
            x_ref, st_ref, o_ref = refs
        i = pl.program_id(0)
        y = jnp.maximum(x_ref[...] * st_ref[0:1, :] + st_ref[1:2, :], 0.0)
        if resid:
            y = y + r_ref[...]
        rows = lax.broadcasted_iota(jnp.int32, (BR, 1), 0) + i * BR
        o_ref[...] = jnp.where(rows < n_true, y, 0.0)

    in_specs = [
        pl.BlockSpec((BR, dout), lambda i: (i, 0)),
        pl.BlockSpec((8, dout), lambda i: (0, 0)),
    ]
    args = [x, st]
    if resid:
        in_specs.append(pl.BlockSpec((BR, dout), lambda i: (i, 0)))
        args.append(res)
    return pl.pallas_call(
        body,
        grid=(n // BR,),
        in_specs=in_specs,
        out_specs=pl.BlockSpec((BR, dout), lambda i: (i, 0)),
        out_shape=jax.ShapeDtypeStruct((n, dout), jnp.float32),
    )(*args)


def _u_layer(u_in, mh, me, Fw, fb, Gw, Hw, g, b, resid):
    din = u_in.shape[1]
    dout = Fw.shape[1]

    def body(u_ref, mh_ref, me_ref, fw_ref, fb_ref, gw_ref, hw_ref,
             gb_ref, o_ref):
        x = (
            jnp.dot(u_ref[...], fw_ref[...], preferred_element_type=jnp.float32)
            + fb_ref[...]
            + jnp.dot(mh_ref[...], gw_ref[...], preferred_element_type=jnp.float32)
            + jnp.dot(me_ref[...], hw_ref[...], preferred_element_type=jnp.float32)
        )
        rows = lax.broadcasted_iota(jnp.int32, (GP, 1), 0)
        mask = rows < NG
        x = jnp.where(mask, x, 0.0)
        mu = jnp.sum(x, axis=0, keepdims=True) / NG
        xc = x - mu
        var = jnp.sum(jnp.where(mask, xc * xc, 0.0), axis=0, keepdims=True) / NG
        y = gb_ref[0:1, :] * xc * jax.lax.rsqrt(var + 1e-5) + gb_ref[1:2, :]
        y = jnp.maximum(y, 0.0)
        if resid:
            y = y + u_ref[...]
        o_ref[...] = jnp.where(mask, y, 0.0)

    gb = jnp.concatenate(
        [g.reshape(1, dout), b.reshape(1, dout),
         jnp.zeros((6, dout), jnp.float32)], axis=0)
    return pl.pallas_call(
        body,
        grid=(1,),
        in_specs=[
            pl.BlockSpec((GP, din), lambda i: (0, 0)),
            pl.BlockSpec((GP, dout), lambda i: (0, 0)),
            pl.BlockSpec((GP, dout), lambda i: (0, 0)),
            pl.BlockSpec((din, dout), lambda i: (0, 0)),
            pl.BlockSpec((1, dout), lambda i: (0, 0)),
            pl.BlockSpec((dout, dout), lambda i: (0, 0)),
            pl.BlockSpec((dout, dout), lambda i: (0, 0)),
            pl.BlockSpec((8, dout), lambda i: (0, 0)),
        ],
        out_specs=pl.BlockSpec((GP, dout), lambda i: (0, 0)),
        out_shape=jax.ShapeDtypeStruct((GP, dout), jnp.float32),
    )(u_in, mh, me, Fw, fb.reshape(1, dout), Gw, Hw, gb)


def _lstm_iter(qs, hs, cs, w0, w1, w2, v0, v1, v2, biases, d):
    def body(qs_ref, hs_ref, cs_ref, w0_ref, w1_ref, w2_ref,
             v0_ref, v1_ref, v2_ref, b_ref, q_ref, ho_ref, co_ref):
        x = qs_ref[...]
        wr = [w0_ref, w1_ref, w2_ref]
        vr = [v0_ref, v1_ref, v2_ref]
        for l in range(3):
            h = hs_ref[:, l * d:(l + 1) * d]
            c = cs_ref[:, l * d:(l + 1) * d]
            gates = (
                jnp.dot(x, wr[l][...], preferred_element_type=jnp.float32)
                + jnp.dot(h, vr[l][...], preferred_element_type=jnp.float32)
                + b_ref[l:l + 1, :]
            )
            ii = gates[:, 0:d]
            ff = gates[:, d:2 * d]
            gg = gates[:, 2 * d:3 * d]
            oo = gates[:, 3 * d:4 * d]
            c2 = jax.nn.sigmoid(ff) * c + jax.nn.sigmoid(ii) * jnp.tanh(gg)
            h2 = jax.nn.sigmoid(oo) * jnp.tanh(c2)
            ho_ref[:, l * d:(l + 1) * d] = h2
            co_ref[:, l * d:(l + 1) * d] = c2
            x = h2
        q_ref[...] = x

    full = lambda shape: pl.BlockSpec(shape, lambda i: tuple(0 for _ in shape))
    return pl.pallas_call(
        body,
        grid=(1,),
        in_specs=[
            full((GP, 2 * d)), full((GP, 3 * d)), full((GP, 3 * d)),
            full((2 * d, 4 * d)), full((d, 4 * d)), full((d, 4 * d)),
            full((d, 4 * d)), full((d, 4 * d)), full((d, 4 * d)),
            full((8, 4 * d)),
        ],
        out_specs=[full((GP, d)), full((GP, 3 * d)), full((GP, 3 * d))],
        out_shape=[
            jax.ShapeDtypeStruct((GP, d), jnp.float32),
            jax.ShapeDtypeStruct((GP, 3 * d), jnp.float32),
            jax.ShapeDtypeStruct((GP, 3 * d), jnp.float32),
        ],
    )(qs, hs, cs, w0, w1, w2, v0, v1, v2, biases)


def _scores(feat, qseg, n_true):
    n, d = feat.shape

    def body(f_ref, q_ref, o_ref, p_ref):
        i = pl.program_id(0)
        s = jnp.sum(f_ref[...] * q_ref[...], axis=1, keepdims=True)
        rows = lax.broadcasted_iota(jnp.int32, (BR, 1), 0) + i * BR
        s = jnp.where(rows < n_true, s, -3.0e38)
        o_ref[...] = s

        @pl.when(i == 0)
        def _():
            p_ref[...] = jnp.full_like(p_ref, -3.0e38)

        m = jnp.max(s)
        p_ref[...] = jnp.maximum(p_ref[...], m)

    return pl.pallas_call(
        body,
        grid=(n // BR,),
        in_specs=[
            pl.BlockSpec((BR, d), lambda i: (i, 0)),
            pl.BlockSpec((BR, d), lambda i: (i, 0)),
        ],
        out_specs=[
            pl.BlockSpec((BR, 1), lambda i: (i, 0)),
            pl.BlockSpec((8, 128), lambda i: (0, 0)),
        ],
        out_shape=[
            jax.ShapeDtypeStruct((n, 1), jnp.float32),
            jax.ShapeDtypeStruct((8, 128), jnp.float32),
        ],
    )(feat, qseg)


def _msgs(feat, scores, gmax):
    n, d = feat.shape

    def body(f_ref, s_ref, g_ref, o_ref):
        ex = jnp.exp(s_ref[...] - g_ref[0:1, 0:1])
        o_ref[:, :d] = f_ref[...] * ex
        o_ref[:, d:d + 16] = jnp.broadcast_to(ex, (BR, 16))

    return pl.pallas_call(
        body,
        grid=(n // BR,),
        in_specs=[
            pl.BlockSpec((BR, d), lambda i: (i, 0)),
            pl.BlockSpec((BR, 1), lambda i: (i, 0)),
            pl.BlockSpec((8, 128), lambda i: (0, 0)),
        ],
        out_specs=pl.BlockSpec((BR, d + 16), lambda i: (i, 0)),
        out_shape=jax.ShapeDtypeStruct((n, d + 16), jnp.float32),
    )(feat, scores, gmax)


def _mlp(x, w0, b0, w1, b1, w2, b2):
    def body(x_ref, w0r, b0r, w1r, b1r, w2r, b2r, o_ref):
        y = jnp.maximum(
            jnp.dot(x_ref[...], w0r[...], preferred_element_type=jnp.float32)
            + b0r[...], 0.0)
        y = jnp.maximum(
            jnp.dot(y, w1r[...], preferred_element_type=jnp.float32)
            + b1r[...], 0.0)
        o_ref[...] = (
            jnp.dot(y, w2r[...], preferred_element_type=jnp.float32) + b2r[...])

    full = lambda shape: pl.BlockSpec(shape, lambda i: tuple(0 for _ in shape))
    return pl.pallas_call(
        body,
        grid=(1,),
        in_specs=[
            full((GP, 160)), full((160, 32)), full((1, 32)),
            full((32, 16)), full((1, 16)), full((16, 1)), full((1, 1)),
        ],
        out_specs=full((GP, 1)),
        out_shape=jax.ShapeDtypeStruct((GP, 1), jnp.float32),
    )(x, w0, b0.reshape(1, 32), w1, b1.reshape(1, 16), w2, b2.reshape(1, 1))


def _sc_gather(table, idx):
    V, D = table.shape
    B, = idx.shape
    NW = NC * NS
    bpw = B // NW
    nch = bpw // CH
    mesh = plsc.VectorSubcoreMesh(core_axis_name="c", subcore_axis_name="s")

    @functools.partial(
        pl.kernel, mesh=mesh,
        out_type=jax.ShapeDtypeStruct((B, D), jnp.float32),
        compiler_params=pltpu.CompilerParams(use_tc_tiling_on_sc=False),
        scratch_types=[
            pltpu.VMEM((CH,), jnp.int32),
            pltpu.VMEM((CH, D), jnp.float32),
            pltpu.SemaphoreType.DMA,
        ],
    )
    def k(table_hbm, idx_hbm, out_hbm, idx_v, rows_v, sem):
        wid = lax.axis_index("s") * NC + lax.axis_index("c")
        base = wid * bpw

        def step(j, carry):
            off = base + j * CH
            pltpu.sync_copy(idx_hbm.at[pl.ds(off, CH)], idx_v)
            pltpu.async_copy(table_hbm.at[idx_v], rows_v, sem).wait()
            pltpu.sync_copy(rows_v, out_hbm.at[pl.ds(off, CH)])
            return carry

        lax.fori_loop(0, nch, step, 0)

    return k(table, idx)


def _sc_scatter_small(idx, msg, nrows):
    B, D = msg.shape
    bpw = B // (NC * NS)
    nch = bpw // CH
    rpt = nrows // NS
    z = jnp.zeros((nrows, D), jnp.float32)
    mesh = plsc.VectorSubcoreMesh(core_axis_name="c", subcore_axis_name="s")

    @functools.partial(
        pl.kernel, mesh=mesh,
        out_type=jax.ShapeDtypeStruct((NC, nrows, D), jnp.float32),
        compiler_params=pltpu.CompilerParams(use_tc_tiling_on_sc=False),
        scratch_types=[
            pltpu.VMEM((CH,), jnp.int32),
            pltpu.VMEM((CH, D), jnp.float32),
            pltpu.VMEM_SHARED((nrows, D), jnp.float32),
        ],
    )
    def k(idx_hbm, msg_hbm, z_hbm, out_hbm, idx_v, msg_v, shared):
        c = lax.axis_index("c")
        s = lax.axis_index("s")
        wid = s * NC + c
        pltpu.sync_copy(z_hbm.at[pl.ds(s * rpt, rpt)],
                        shared.at[pl.ds(s * rpt, rpt)])
        plsc.subcore_barrier()
        base = wid * bpw

        def step(j, carry):
            off = base + j * CH
            pltpu.sync_copy(idx_hbm.at[pl.ds(off, CH)], idx_v)
            pltpu.sync_copy(msg_hbm.at[pl.ds(off, CH)], msg_v)
            pltpu.sync_copy(msg_v, shared.at[idx_v], add=True)
            return carry

        lax.fori_loop(0, nch, step, 0)
        plsc.subcore_barrier()
        pltpu.sync_copy(shared.at[pl.ds(s * rpt, rpt)],
                        out_hbm.at[c, pl.ds(s * rpt, rpt)])

    return k(idx, msg, z)


def _sc_scatter_atom(idx_a, msg_a, idx_b, msg_b, nrows_out):
    B, D = msg_a.shape
    half = D // 2
    F = half // 16
    bpt = B // NS
    nch = bpt // CH
    rpt = NA // NS
    z = jnp.zeros((NA, 16), jnp.float32)
    mesh = plsc.VectorSubcoreMesh(core_axis_name="c", subcore_axis_name="s")

    @functools.partial(
        pl.kernel, mesh=mesh,
        out_type=jax.ShapeDtypeStruct((nrows_out, D), jnp.float32),
        compiler_params=pltpu.CompilerParams(use_tc_tiling_on_sc=False),
        scratch_types=[
            pltpu.VMEM((CH,), jnp.int32),
            pltpu.VMEM((CH, 16), jnp.float32),
            pltpu.VMEM_SHARED((NA, 16), jnp.float32),
        ],
    )
    def k(ia_hbm, ma_hbm, ib_hbm, mb_hbm, z_hbm, out_hbm, idx_v, msg_v, shared):
        c = lax.axis_index("c")
        s = lax.axis_index("s")
        for f in range(F):
            fcol = c * half + f * 16
            pltpu.sync_copy(z_hbm.at[pl.ds(s * rpt, rpt)],
                            shared.at[pl.ds(s * rpt, rpt)])
            plsc.subcore_barrier()
            for ih, mh in ((ia_hbm, ma_hbm), (ib_hbm, mb_hbm)):
                def step(j, carry, ih=ih, mh=mh):
                    off = s * bpt + j * CH
                    pltpu.sync_copy(ih.at[pl.ds(off, CH)], idx_v)
                    pltpu.sync_copy(mh.at[pl.ds(off, CH), pl.ds(fcol, 16)],
                                    msg_v)
                    pltpu.sync_copy(msg_v, shared.at[idx_v], add=True)
                    return carry

                lax.fori_loop(0, nch, step, 0)
            plsc.subcore_barrier()
            pltpu.sync_copy(shared.at[pl.ds(s * rpt, rpt)],
                            out_hbm.at[pl.ds(s * rpt, rpt), pl.ds(fcol, 16)])

    return k(idx_a, msg_a, idx_b, msg_b, z)


def _pad_rows(x, n):
    return jnp.pad(x, ((0, n - x.shape[0]),) + ((0, 0),) * (x.ndim - 1))


def _bn_scale_shift(part, g, b, n_true):
    mu = part[0] / n_true
    var = jnp.maximum(part[1] / n_true - mu * mu, 0.0)
    s = g / jnp.sqrt(var + 1e-5)
    t = b - mu * s
    dout = s.shape[0]
    return jnp.concatenate(
        [s.reshape(1, dout), t.reshape(1, dout),
         jnp.zeros((6, dout), jnp.float32)], axis=0)


def _set2set(feat, seg, P, prefix, n_true):
    d = feat.shape[1]
    q_star = jnp.zeros((GP, 2 * d), jnp.float32)
    hs = jnp.zeros((GP, 3 * d), jnp.float32)
    cs = jnp.zeros((GP, 3 * d), jnp.float32)
    w0 = P[prefix + "_Wih0"].T
    w1 = P[prefix + "_Wih1"].T
    w2 = P[prefix + "_Wih2"].T
    v0 = P[prefix + "_Whh0"].T
    v1 = P[prefix + "_Whh1"].T
    v2 = P[prefix + "_Whh2"].T
    biases = jnp.concatenate(
        [P[prefix + "_bias0"].reshape(1, 4 * d),
         P[prefix + "_bias1"].reshape(1, 4 * d),
         P[prefix + "_bias2"].reshape(1, 4 * d),
         jnp.zeros((5, 4 * d), jnp.float32)], axis=0)
    for _ in range(5):
        q, hs, cs = _lstm_iter(q_star, hs, cs, w0, w1, w2, v0, v1, v2,
                               biases, d)
        qseg = _sc_gather(q, seg)
        scores, gmax = _scores(feat, qseg, n_true)
        m = _msgs(feat, scores, gmax)
        S = jnp.sum(_sc_scatter_small(seg, m, NG), axis=0)
        r = S[:, :d] / (S[:, d:d + 1] + 1e-12)
        q_star = jnp.concatenate([q[:NG], r], axis=1)
        q_star = _pad_rows(q_star, GP)
    return q_star[:NG]


def kernel(atom_feats, bond_feats, global_feats, edge_index, atom2graph, bond2graph, norm_atom, norm_bond, emb_atom_w, emb_bond_w, emb_global_w, L0_A_w, L0_A_b, L0_B_w, L0_B_b, L0_C_w, L0_C_b, L0_D_w, L0_D_b, L0_E_w, L0_E_b, L0_F_w, L0_F_b, L0_G_w, L0_H_w, L0_bn_h_g, L0_bn_h_b, L0_bn_e_g, L0_bn_e_b, L0_bn_u_g, L0_bn_u_b, L1_A_w, L1_A_b, L1_B_w, L1_B_b, L1_C_w, L1_C_b, L1_D_w, L1_D_b, L1_E_w, L1_E_b, L1_F_w, L1_F_b, L1_G_w, L1_H_w, L1_bn_h_g, L1_bn_h_b, L1_bn_e_g, L1_bn_e_b, L1_bn_u_g, L1_bn_u_b, L2_A_w, L2_A_b, L2_B_w, L2_B_b, L2_C_w, L2_C_b, L2_D_w, L2_D_b, L2_E_w, L2_E_b, L2_F_w, L2_F_b, L2_G_w, L2_H_w, L2_bn_h_g, L2_bn_h_b, L2_bn_e_g, L2_bn_e_b, L2_bn_u_g, L2_bn_u_b, s2s_atom_Wih0, s2s_atom_Whh0, s2s_atom_bias0, s2s_atom_Wih1, s2s_atom_Whh1, s2s_atom_bias1, s2s_atom_Wih2, s2s_atom_Whh2, s2s_atom_bias2, s2s_bond_Wih0, s2s_bond_Whh0, s2s_bond_bias0, s2s_bond_Wih1, s2s_bond_Whh1, s2s_bond_bias1, s2s_bond_Wih2, s2s_bond_Whh2, s2s_bond_bias2, fc0_w, fc0_b, fc1_w, fc1_b, fc2_w, fc2_b):
    P = dict(locals())
    src = _pad_rows(edge_index[0].reshape(NB, 1), BP)[:, 0]
    dst = _pad_rows(edge_index[1].reshape(NB, 1), BP)[:, 0]
    a2g = _pad_rows(atom2graph.reshape(NA, 1), AP)[:, 0]
    b2g = _pad_rows(bond2graph.reshape(NB, 1), BP)[:, 0]
    na_ = _pad_rows(norm_atom, AP)
    nb_ = _pad_rows(norm_bond, BP)

    h = _mm(_pad_rows(atom_feats, AP), emb_atom_w)
    e = _mm(_pad_rows(bond_feats, BP), emb_bond_w)
    u = _mm(_pad_rows(global_feats, GP), emb_global_w)

    dims = [32, 64, 64, 32]
    ones_a = (jnp.arange(AP) < NA).astype(jnp.float32)[:, None] * jnp.ones(
        (1, 16), jnp.float32)
    ones_b = (jnp.arange(BP) < NB).astype(jnp.float32)[:, None] * jnp.ones(
        (1, 16), jnp.float32)
    ca = jnp.maximum(jnp.sum(_sc_scatter_small(a2g, ones_a, NG), axis=0)[:, :1],
                     1.0)
    cb = jnp.maximum(jnp.sum(_sc_scatter_small(b2g, ones_b, NG), axis=0)[:, :1],
                     1.0)

    for i in range(3):
        din, dout = dims[i], dims[i + 1]
        Aw = P["L%d_A_w" % i]; Ab = P["L%d_A_b" % i]
        Bw = P["L%d_B_w" % i]; Bb = P["L%d_B_b" % i]
        Cw = P["L%d_C_w" % i]; Cb = P["L%d_C_b" % i]
        Dw = P["L%d_D_w" % i]; Db = P["L%d_D_b" % i]
        Ew = P["L%d_E_w" % i]; Eb = P["L%d_E_b" % i]
        Fw = P["L%d_F_w" % i]; Fb = P["L%d_F_b" % i]

        ah = _mm(h, Aw, Ab)
        bd = _mm(h, jnp.concatenate([Bw, Dw], axis=1),
                 jnp.concatenate([Bb, jnp.zeros_like(Db)], axis=0))
        eu_tbl = _mm(u, Ew)
        gs = _sc_gather(bd, src)
        gd = _sc_gather(bd, dst)
        eu = _sc_gather(eu_tbl, b2g)

        epre, pe = _edge_pre(e, gs, gd, eu, nb_, Cw,
                             (Cb + Db + Eb).reshape(1, dout), NB)
        st_e = _bn_scale_shift(pe, P["L%d_bn_e_g" % i], P["L%d_bn_e_b" % i], NB)
        e_new, sig, m1, m2 = _edge_post(epre, gs, gd, st_e,
                                        e if din == dout else None, NB)
        num = _sc_scatter_atom(src, m1, dst, m2, AP)
        den = _sc_scatter_atom(src, sig, dst, sig, AP)
        hpre, ph = _atom_pre(ah, num, den, na_, NA)
        st_h = _bn_scale_shift(ph, P["L%d_bn_h_g" % i], P["L%d_bn_h_b" % i], NA)
        h_new = _post_bn_relu(hpre, st_h, h if din == dout else None, NA)

        Sh = jnp.sum(_sc_scatter_small(a2g, h_new, NG), axis=0)
        Se = jnp.sum(_sc_scatter_small(b2g, e_new, NG), axis=0)
        mh = _pad_rows(Sh / ca, GP)
        me = _pad_rows(Se / cb, GP)
        u = _u_layer(u, mh, me, Fw, Fb, P["L%d_G_w" % i], P["L%d_H_w" % i],
                     P["L%d_bn_u_g" % i], P["L%d_bn_u_b" % i], din == dout)
        h, e = h_new, e_new

    s_a = _set2set(h, a2g, P, "s2s_atom", NA)
    s_b = _set2set(e, b2g, P, "s2s_bond", NB)
    x = jnp.concatenate([s_a, s_b, u[:NG]], axis=1)
    out = _mlp(_pad_rows(x, GP), fc0_w, fc0_b, fc1_w, fc1_b, fc2_w, fc2_b)
    return out[:NG]


# speedup vs baseline: 4.4749x; 1.3277x over previous
import functools

import jax
import jax.numpy as jnp
from jax import lax
from jax.experimental import pallas as pl
from jax.experimental.pallas import tpu as pltpu
from jax.experimental.pallas import tpu_sc as plsc

NA = 50000
NB = 800000
NG = 2000
BR = 2048
BP = 802816
AP = 53248
GP = 2048
NC = 2
NS = 16
CH = 128


def _mm(x, w, b=None):
    n, din = x.shape
    dout = w.shape[1]
    bias = jnp.zeros((1, dout), jnp.float32) if b is None else b.reshape(1, dout)

    def body(x_ref, w_ref, b_ref, o_ref):
        o_ref[...] = (
            jnp.dot(x_ref[...], w_ref[...], preferred_element_type=jnp.float32)
            + b_ref[...]
        )

    return pl.pallas_call(
        body,
        grid=(n // BR,),
        in_specs=[
            pl.BlockSpec((BR, din), lambda i: (i, 0)),
            pl.BlockSpec((din, dout), lambda i: (0, 0)),
            pl.BlockSpec((1, dout), lambda i: (0, 0)),
        ],
        out_specs=pl.BlockSpec((BR, dout), lambda i: (i, 0)),
        out_shape=jax.ShapeDtypeStruct((n, dout), jnp.float32),
    )(x, w, bias)


def _edge_pre(e, gs, gd, eu, nb_, w, bias, n_true):
    n, din = e.shape
    dout = w.shape[1]

    def body(e_ref, gs_ref, gd_ref, eu_ref, nb_ref, w_ref, b_ref, o_ref, p_ref):
        i = pl.program_id(0)
        t = (
            jnp.dot(e_ref[...], w_ref[...], preferred_element_type=jnp.float32)
            + b_ref[...]
        )
        t = t + gs_ref[:, dout:] + gd_ref[:, dout:] + eu_ref[...]
        t = t * nb_ref[...]
        rows = lax.broadcasted_iota(jnp.int32, (BR, 1), 0) + i * BR
        t = jnp.where(rows < n_true, t, 0.0)
        o_ref[...] = t

        @pl.when(i == 0)
        def _():
            p_ref[...] = jnp.zeros_like(p_ref)

        p_ref[0:1, :] += jnp.sum(t, axis=0, keepdims=True)
        p_ref[1:2, :] += jnp.sum(t * t, axis=0, keepdims=True)

    return pl.pallas_call(
        body,
        grid=(n // BR,),
        in_specs=[
            pl.BlockSpec((BR, din), lambda i: (i, 0)),
            pl.BlockSpec((BR, 2 * dout), lambda i: (i, 0)),
            pl.BlockSpec((BR, 2 * dout), lambda i: (i, 0)),
            pl.BlockSpec((BR, dout), lambda i: (i, 0)),
            pl.BlockSpec((BR, 1), lambda i: (i, 0)),
            pl.BlockSpec((din, dout), lambda i: (0, 0)),
            pl.BlockSpec((1, dout), lambda i: (0, 0)),
        ],
        out_specs=[
            pl.BlockSpec((BR, dout), lambda i: (i, 0)),
            pl.BlockSpec((8, dout), lambda i: (0, 0)),
        ],
        out_shape=[
            jax.ShapeDtypeStruct((n, dout), jnp.float32),
            jax.ShapeDtypeStruct((8, dout), jnp.float32),
        ],
    )(e, gs, gd, eu, nb_, w, bias)


def _edge_post(epre, gs, gd, st, e_res, n_true):
    n, dout = epre.shape
    resid = e_res is not None

    def body(*refs):
        if resid:
            ep_ref, gs_ref, gd_ref, st_ref, er_ref = refs[:5]
            outs = refs[5:]
        else:
            ep_ref, gs_ref, gd_ref, st_ref = refs[:4]
            outs = refs[4:]
        en_ref, sg_ref, m1_ref, m2_ref = outs
        i = pl.program_id(0)
        x = ep_ref[...] * st_ref[0:1, :] + st_ref[1:2, :]
        x = jnp.maximum(x, 0.0)
        if resid:
            x = x + er_ref[...]
        rows = lax.broadcasted_iota(jnp.int32, (BR, 1), 0) + i * BR
        mask = rows < n_true
        x = jnp.where(mask, x, 0.0)
        sg = jnp.where(mask, jax.nn.sigmoid(x), 0.0)
        en_ref[...] = x
        sg_ref[...] = sg
        m1_ref[...] = sg * gd_ref[:, :dout]
        m2_ref[...] = sg * gs_ref[:, :dout]

    in_specs = [
        pl.BlockSpec((BR, dout), lambda i: (i, 0)),
        pl.BlockSpec((BR, 2 * dout), lambda i: (i, 0)),
        pl.BlockSpec((BR, 2 * dout), lambda i: (i, 0)),
        pl.BlockSpec((8, dout), lambda i: (0, 0)),
    ]
    args = [epre, gs, gd, st]
    if resid:
        in_specs.append(pl.BlockSpec((BR, dout), lambda i: (i, 0)))
        args.append(e_res)
    sh = jax.ShapeDtypeStruct((n, dout), jnp.float32)
    return pl.pallas_call(
        body,
        grid=(n // BR,),
        in_specs=in_specs,
        out_specs=[pl.BlockSpec((BR, dout), lambda i: (i, 0))] * 4,
        out_shape=[sh, sh, sh, sh],
    )(*args)


def _atom_pre(ah, num, den, na_, n_true):
    n, dout = ah.shape

    def body(a_ref, nu_ref, de_ref, na_ref, o_ref, p_ref):
        i = pl.program_id(0)
        t = a_ref[...] + nu_ref[...] / (de_ref[...] + 1e-6)
        t = t * na_ref[...]
        rows = lax.broadcasted_iota(jnp.int32, (BR, 1), 0) + i * BR
        t = jnp.where(rows < n_true, t, 0.0)
        o_ref[...] = t

        @pl.when(i == 0)
        def _():
            p_ref[...] = jnp.zeros_like(p_ref)

        p_ref[0:1, :] += jnp.sum(t, axis=0, keepdims=True)
        p_ref[1:2, :] += jnp.sum(t * t, axis=0, keepdims=True)

    return pl.pallas_call(
        body,
        grid=(n // BR,),
        in_specs=[
            pl.BlockSpec((BR, dout), lambda i: (i, 0)),
            pl.BlockSpec((BR, dout), lambda i: (i, 0)),
            pl.BlockSpec((BR, dout), lambda i: (i, 0)),
            pl.BlockSpec((BR, 1), lambda i: (i, 0)),
        ],
        out_specs=[
            pl.BlockSpec((BR, dout), lambda i: (i, 0)),
            pl.BlockSpec((8, dout), lambda i: (0, 0)),
        ],
        out_shape=[
            jax.ShapeDtypeStruct((n, dout), jnp.float32),
            jax.ShapeDtypeStruct((8, dout), jnp.float32),
        ],
    )(ah, num, den, na_)


def _post_bn_relu(x, st, res, n_true):
    n, dout = x.shape
    resid = res is not None

    def body(*refs):
        if resid:
            x_ref, st_ref, r_ref, o_ref = refs
        else:
            x_ref, st_ref, o_ref = refs
        i = pl.program_id(0)
        y = jnp.maximum(x_ref[...] * st_ref[0:1, :] + st_ref[1:2, :], 0.0)
        if resid:
            y = y + r_ref[...]
        rows = lax.broadcasted_iota(jnp.int32, (BR, 1), 0) + i * BR
        o_ref[...] = jnp.where(rows < n_true, y, 0.0)

    in_specs = [
        pl.BlockSpec((BR, dout), lambda i: (i, 0)),
        pl.BlockSpec((8, dout), lambda i: (0, 0)),
    ]
    args = [x, st]
    if resid:
        in_specs.append(pl.BlockSpec((BR, dout), lambda i: (i, 0)))
        args.append(res)
    return pl.pallas_call(
        body,
        grid=(n // BR,),
        in_specs=in_specs,
        out_specs=pl.BlockSpec((BR, dout), lambda i: (i, 0)),
        out_shape=jax.ShapeDtypeStruct((n, dout), jnp.float32),
    )(*args)


def _u_layer(u_in, mh, me, Fw, fb, Gw, Hw, g, b, resid):
    din = u_in.shape[1]
    dout = Fw.shape[1]

    def body(u_ref, mh_ref, me_ref, fw_ref, fb_ref, gw_ref, hw_ref,
             gb_ref, o_ref):
        x = (
            jnp.dot(u_ref[...], fw_ref[...], preferred_element_type=jnp.float32)
            + fb_ref[...]
            + jnp.dot(mh_ref[...], gw_ref[...], preferred_element_type=jnp.float32)
            + jnp.dot(me_ref[...], hw_ref[...], preferred_element_type=jnp.float32)
        )
        rows = lax.broadcasted_iota(jnp.int32, (GP, 1), 0)
        mask = rows < NG
        x = jnp.where(mask, x, 0.0)
        mu = jnp.sum(x, axis=0, keepdims=True) / NG
        xc = x - mu
        var = jnp.sum(jnp.where(mask, xc * xc, 0.0), axis=0, keepdims=True) / NG
        y = gb_ref[0:1, :] * xc * jax.lax.rsqrt(var + 1e-5) + gb_ref[1:2, :]
        y = jnp.maximum(y, 0.0)
        if resid:
            y = y + u_ref[...]
        o_ref[...] = jnp.where(mask, y, 0.0)

    gb = jnp.concatenate(
        [g.reshape(1, dout), b.reshape(1, dout),
         jnp.zeros((6, dout), jnp.float32)], axis=0)
    return pl.pallas_call(
        body,
        grid=(1,),
        in_specs=[
            pl.BlockSpec((GP, din), lambda i: (0, 0)),
            pl.BlockSpec((GP, dout), lambda i: (0, 0)),
            pl.BlockSpec((GP, dout), lambda i: (0, 0)),
            pl.BlockSpec((din, dout), lambda i: (0, 0)),
            pl.BlockSpec((1, dout), lambda i: (0, 0)),
            pl.BlockSpec((dout, dout), lambda i: (0, 0)),
            pl.BlockSpec((dout, dout), lambda i: (0, 0)),
            pl.BlockSpec((8, dout), lambda i: (0, 0)),
        ],
        out_specs=pl.BlockSpec((GP, dout), lambda i: (0, 0)),
        out_shape=jax.ShapeDtypeStruct((GP, dout), jnp.float32),
    )(u_in, mh, me, Fw, fb.reshape(1, dout), Gw, Hw, gb)


def _lstm_iter(qs, hs, cs, w0, w1, w2, v0, v1, v2, biases, d):
    def body(qs_ref, hs_ref, cs_ref, w0_ref, w1_ref, w2_ref,
             v0_ref, v1_ref, v2_ref, b_ref, q_ref, ho_ref, co_ref):
        x = qs_ref[...]
        wr = [w0_ref, w1_ref, w2_ref]
        vr = [v0_ref, v1_ref, v2_ref]
        for l in range(3):
            h = hs_ref[:, l * d:(l + 1) * d]
            c = cs_ref[:, l * d:(l + 1) * d]
            gates = (
                jnp.dot(x, wr[l][...], preferred_element_type=jnp.float32)
                + jnp.dot(h, vr[l][...], preferred_element_type=jnp.float32)
                + b_ref[l:l + 1, :]
            )
            ii = gates[:, 0:d]
            ff = gates[:, d:2 * d]
            gg = gates[:, 2 * d:3 * d]
            oo = gates[:, 3 * d:4 * d]
            c2 = jax.nn.sigmoid(ff) * c + jax.nn.sigmoid(ii) * jnp.tanh(gg)
            h2 = jax.nn.sigmoid(oo) * jnp.tanh(c2)
            ho_ref[:, l * d:(l + 1) * d] = h2
            co_ref[:, l * d:(l + 1) * d] = c2
            x = h2
        q_ref[...] = x

    full = lambda shape: pl.BlockSpec(shape, lambda i: tuple(0 for _ in shape))
    return pl.pallas_call(
        body,
        grid=(1,),
        in_specs=[
            full((GP, 2 * d)), full((GP, 3 * d)), full((GP, 3 * d)),
            full((2 * d, 4 * d)), full((d, 4 * d)), full((d, 4 * d)),
            full((d, 4 * d)), full((d, 4 * d)), full((d, 4 * d)),
            full((8, 4 * d)),
        ],
        out_specs=[full((GP, d)), full((GP, 3 * d)), full((GP, 3 * d))],
        out_shape=[
            jax.ShapeDtypeStruct((GP, d), jnp.float32),
            jax.ShapeDtypeStruct((GP, 3 * d), jnp.float32),
            jax.ShapeDtypeStruct((GP, 3 * d), jnp.float32),
        ],
    )(qs, hs, cs, w0, w1, w2, v0, v1, v2, biases)


def _scores(feat, qseg, n_true):
    n, d = feat.shape

    def body(f_ref, q_ref, o_ref, p_ref):
        i = pl.program_id(0)
        s = jnp.sum(f_ref[...] * q_ref[...], axis=1, keepdims=True)
        rows = lax.broadcasted_iota(jnp.int32, (BR, 1), 0) + i * BR
        s = jnp.where(rows < n_true, s, -3.0e38)
        o_ref[...] = s

        @pl.when(i == 0)
        def _():
            p_ref[...] = jnp.full_like(p_ref, -3.0e38)

        m = jnp.max(s)
        p_ref[...] = jnp.maximum(p_ref[...], m)

    return pl.pallas_call(
        body,
        grid=(n // BR,),
        in_specs=[
            pl.BlockSpec((BR, d), lambda i: (i, 0)),
            pl.BlockSpec((BR, d), lambda i: (i, 0)),
        ],
        out_specs=[
            pl.BlockSpec((BR, 1), lambda i: (i, 0)),
            pl.BlockSpec((8, 128), lambda i: (0, 0)),
        ],
        out_shape=[
            jax.ShapeDtypeStruct((n, 1), jnp.float32),
            jax.ShapeDtypeStruct((8, 128), jnp.float32),
        ],
    )(feat, qseg)


def _msgs(feat, scores, gmax):
    n, d = feat.shape

    def body(f_ref, s_ref, g_ref, o_ref):
        ex = jnp.exp(s_ref[...] - g_ref[0:1, 0:1])
        o_ref[:, :d] = f_ref[...] * ex
        o_ref[:, d:d + 16] = jnp.broadcast_to(ex, (BR, 16))

    return pl.pallas_call(
        body,
        grid=(n // BR,),
        in_specs=[
            pl.BlockSpec((BR, d), lambda i: (i, 0)),
            pl.BlockSpec((BR, 1), lambda i: (i, 0)),
            pl.BlockSpec((8, 128), lambda i: (0, 0)),
        ],
        out_specs=pl.BlockSpec((BR, d + 16), lambda i: (i, 0)),
        out_shape=jax.ShapeDtypeStruct((n, d + 16), jnp.float32),
    )(feat, scores, gmax)


def _mlp(x, w0, b0, w1, b1, w2, b2):
    def body(x_ref, w0r, b0r, w1r, b1r, w2r, b2r, o_ref):
        y = jnp.maximum(
            jnp.dot(x_ref[...], w0r[...], preferred_element_type=jnp.float32)
            + b0r[...], 0.0)
        y = jnp.maximum(
            jnp.dot(y, w1r[...], preferred_element_type=jnp.float32)
            + b1r[...], 0.0)
        o_ref[...] = (
            jnp.dot(y, w2r[...], preferred_element_type=jnp.float32) + b2r[...])

    full = lambda shape: pl.BlockSpec(shape, lambda i: tuple(0 for _ in shape))
    return pl.pallas_call(
        body,
        grid=(1,),
        in_specs=[
            full((GP, 160)), full((160, 32)), full((1, 32)),
            full((32, 16)), full((1, 16)), full((16, 1)), full((1, 1)),
        ],
        out_specs=full((GP, 1)),
        out_shape=jax.ShapeDtypeStruct((GP, 1), jnp.float32),
    )(x, w0, b0.reshape(1, 32), w1, b1.reshape(1, 16), w2, b2.reshape(1, 1))


def _pick_k(nch, bytes_per_chunk):
    for k in range(16, 0, -1):
        if nch % k == 0 and k * bytes_per_chunk <= 400_000:
            return k
    return 1


def _sc_gather(table, idx2d_list, B):
    V, D = table.shape
    NW = NC * NS
    nch = B // NW // CH
    K = _pick_k(nch, CH * D * 4)
    nsi = nch // K
    nstream = len(idx2d_list)
    mesh = plsc.VectorSubcoreMesh(core_axis_name="c", subcore_axis_name="s")
    outs = tuple(jax.ShapeDtypeStruct((B, D), jnp.float32)
                 for _ in range(nstream))

    @functools.partial(
        pl.kernel, mesh=mesh,
        out_type=outs,
        compiler_params=pltpu.CompilerParams(use_tc_tiling_on_sc=False),
        scratch_types=[
            pltpu.VMEM((K, CH), jnp.int32),
            pltpu.VMEM((K * CH, D), jnp.float32),
            pltpu.SemaphoreType.DMA,
        ],
    )
    def k_fn(*refs):
        idx_refs = refs[1:1 + nstream]
        out_refs = refs[1 + nstream:1 + 2 * nstream]
        table_hbm = refs[0]
        idx_v, rows_v, sem = refs[1 + 2 * nstream:]
        wid = lax.axis_index("s") * NC + lax.axis_index("c")
        crow0 = wid * nch
        for ih, oh in zip(idx_refs, out_refs):
            def super_step(si, carry, ih=ih, oh=oh):
                crow = crow0 + si * K
                pltpu.sync_copy(ih.at[pl.ds(crow, K)], idx_v)
                hs = []
                for b in range(K):
                    hs.append(pltpu.async_copy(
                        table_hbm.at[idx_v.at[b]],
                        rows_v.at[pl.ds(b * CH, CH)], sem))
                for h in hs:
                    h.wait()
                pltpu.sync_copy(rows_v, oh.at[pl.ds(crow * CH, K * CH)])
                return carry

            lax.fori_loop(0, nsi, super_step, 0)

    res = k_fn(table, *idx2d_list)
    return tuple(res) if isinstance(res, (tuple, list)) else (res,)


def _sc_scatter_small(idx2d, msg, nrows):
    B, D = msg.shape
    nch = B // (NC * NS) // CH
    K = _pick_k(nch, CH * D * 4)
    nsi = nch // K
    rpt = nrows // NS
    z = jnp.zeros((nrows, D), jnp.float32)
    mesh = plsc.VectorSubcoreMesh(core_axis_name="c", subcore_axis_name="s")

    @functools.partial(
        pl.kernel, mesh=mesh,
        out_type=jax.ShapeDtypeStruct((NC, nrows, D), jnp.float32),
        compiler_params=pltpu.CompilerParams(use_tc_tiling_on_sc=False),
        scratch_types=[
            pltpu.VMEM((K, CH), jnp.int32),
            pltpu.VMEM((K * CH, D), jnp.float32),
            pltpu.VMEM_SHARED((nrows, D), jnp.float32),
            pltpu.SemaphoreType.DMA,
        ],
    )
    def k(idx_hbm, msg_hbm, z_hbm, out_hbm, idx_v, msg_v, shared, sem):
        c = lax.axis_index("c")
        s = lax.axis_index("s")
        wid = s * NC + c
        pltpu.sync_copy(z_hbm.at[pl.ds(s * rpt, rpt)],
                        shared.at[pl.ds(s * rpt, rpt)])
        plsc.subcore_barrier()
        crow0 = wid * nch

        def super_step(si, carry):
            crow = crow0 + si * K
            pltpu.sync_copy(idx_hbm.at[pl.ds(crow, K)], idx_v)
            pltpu.sync_copy(msg_hbm.at[pl.ds(crow * CH, K * CH)], msg_v)
            hs = []
            for b in range(K):
                hs.append(pltpu.async_copy(
                    msg_v.at[pl.ds(b * CH, CH)],
                    shared.at[idx_v.at[b]], sem, add=True))
            for h in hs:
                h.wait()
            return carry

        lax.fori_loop(0, nsi, super_step, 0)
        plsc.subcore_barrier()
        pltpu.sync_copy(shared.at[pl.ds(s * rpt, rpt)],
                        out_hbm.at[c, pl.ds(s * rpt, rpt)])

    return k(idx2d, msg, z)


def _sc_scatter_atom(idx2a, m1, idx2b, m2, sig, nrows_out):
    B, D = m1.shape
    half = D // 2
    F = half // 16
    nch = B // NS // CH
    K = _pick_k(nch, CH * 16 * 4)
    nsi = nch // K
    rpt = NA // NS
    z = jnp.zeros((NA, 16), jnp.float32)
    mesh = plsc.VectorSubcoreMesh(core_axis_name="c", subcore_axis_name="s")

    @functools.partial(
        pl.kernel, mesh=mesh,
        out_type=(jax.ShapeDtypeStruct((nrows_out, D), jnp.float32),
                  jax.ShapeDtypeStruct((nrows_out, D), jnp.float32)),
        compiler_params=pltpu.CompilerParams(use_tc_tiling_on_sc=False),
        scratch_types=[
            pltpu.VMEM((K, CH), jnp.int32),
            pltpu.VMEM((K * CH, 16), jnp.float32),
            pltpu.VMEM_SHARED((NA, 16), jnp.float32),
            pltpu.VMEM_SHARED((NA, 16), jnp.float32),
            pltpu.SemaphoreType.DMA,
        ],
    )
    def k(ia_hbm, m1_hbm, ib_hbm, m2_hbm, sg_hbm, z_hbm, num_hbm, den_hbm,
          idx_v, msg_v, sh_num, sh_den, sem):
        c = lax.axis_index("c")
        s = lax.axis_index("s")
        for f in range(F):
            fcol = c * half + f * 16
            pltpu.sync_copy(z_hbm.at[pl.ds(s * rpt, rpt)],
                            sh_num.at[pl.ds(s * rpt, rpt)])
            pltpu.sync_copy(z_hbm.at[pl.ds(s * rpt, rpt)],
                            sh_den.at[pl.ds(s * rpt, rpt)])
            plsc.subcore_barrier()
            for ih, pairs in ((ia_hbm, ((m1_hbm, sh_num), (sg_hbm, sh_den))),
                              (ib_hbm, ((m2_hbm, sh_num), (sg_hbm, sh_den)))):
                def super_step(si, carry, ih=ih, pairs=pairs):
                    crow = s * nch + si * K
                    pltpu.sync_copy(ih.at[pl.ds(crow, K)], idx_v)
                    for mh, tbl in pairs:
                        pltpu.sync_copy(
                            mh.at[pl.ds(crow * CH, K * CH), pl.ds(fcol, 16)],
                            msg_v)
                        hs = []
                        for b in range(K):
                            hs.append(pltpu.async_copy(
                                msg_v.at[pl.ds(b * CH, CH)],
                                tbl.at[idx_v.at[b]], sem, add=True))
                        for h in hs:
                            h.wait()
                    return carry

                lax.fori_loop(0, nsi, super_step, 0)
            plsc.subcore_barrier()
            pltpu.sync_copy(sh_num.at[pl.ds(s * rpt, rpt)],
                            num_hbm.at[pl.ds(s * rpt, rpt), pl.ds(fcol, 16)])
            pltpu.sync_copy(sh_den.at[pl.ds(s * rpt, rpt)],
                            den_hbm.at[pl.ds(s * rpt, rpt), pl.ds(fcol, 16)])

    return k(idx2a, m1, idx2b, m2, sig, z)


def _pad_rows(x, n):
    return jnp.pad(x, ((0, n - x.shape[0]),) + ((0, 0),) * (x.ndim - 1))


def _bn_scale_shift(part, g, b, n_true):
    mu = part[0] / n_true
    var = jnp.maximum(part[1] / n_true - mu * mu, 0.0)
    s = g / jnp.sqrt(var + 1e-5)
    t = b - mu * s
    dout = s.shape[0]
    return jnp.concatenate(
        [s.reshape(1, dout), t.reshape(1, dout),
         jnp.zeros((6, dout), jnp.float32)], axis=0)


def _set2set(feat, seg, P, prefix, n_true):
    d = feat.shape[1]
    q_star = jnp.zeros((GP, 2 * d), jnp.float32)
    hs = jnp.zeros((GP, 3 * d), jnp.float32)
    cs = jnp.zeros((GP, 3 * d), jnp.float32)
    w0 = P[prefix + "_Wih0"].T
    w1 = P[prefix + "_Wih1"].T
    w2 = P[prefix + "_Wih2"].T
    v0 = P[prefix + "_Whh0"].T
    v1 = P[prefix + "_Whh1"].T
    v2 = P[prefix + "_Whh2"].T
    biases = jnp.concatenate(
        [P[prefix + "_bias0"].reshape(1, 4 * d),
         P[prefix + "_bias1"].reshape(1, 4 * d),
         P[prefix + "_bias2"].reshape(1, 4 * d),
         jnp.zeros((5, 4 * d), jnp.float32)], axis=0)
    B = seg.shape[0] * CH
    for _ in range(5):
        q, hs, cs = _lstm_iter(q_star, hs, cs, w0, w1, w2, v0, v1, v2,
                               biases, d)
        qseg, = _sc_gather(q, [seg], B)
        scores, gmax = _scores(feat, qseg, n_true)
        m = _msgs(feat, scores, gmax)
        S = jnp.sum(_sc_scatter_small(seg, m, NG), axis=0)
        r = S[:, :d] / (S[:, d:d + 1] + 1e-12)
        q_star = jnp.concatenate([q[:NG], r], axis=1)
        q_star = _pad_rows(q_star, GP)
    return q_star[:NG]


def kernel(atom_feats, bond_feats, global_feats, edge_index, atom2graph, bond2graph, norm_atom, norm_bond, emb_atom_w, emb_bond_w, emb_global_w, L0_A_w, L0_A_b, L0_B_w, L0_B_b, L0_C_w, L0_C_b, L0_D_w, L0_D_b, L0_E_w, L0_E_b, L0_F_w, L0_F_b, L0_G_w, L0_H_w, L0_bn_h_g, L0_bn_h_b, L0_bn_e_g, L0_bn_e_b, L0_bn_u_g, L0_bn_u_b, L1_A_w, L1_A_b, L1_B_w, L1_B_b, L1_C_w, L1_C_b, L1_D_w, L1_D_b, L1_E_w, L1_E_b, L1_F_w, L1_F_b, L1_G_w, L1_H_w, L1_bn_h_g, L1_bn_h_b, L1_bn_e_g, L1_bn_e_b, L1_bn_u_g, L1_bn_u_b, L2_A_w, L2_A_b, L2_B_w, L2_B_b, L2_C_w, L2_C_b, L2_D_w, L2_D_b, L2_E_w, L2_E_b, L2_F_w, L2_F_b, L2_G_w, L2_H_w, L2_bn_h_g, L2_bn_h_b, L2_bn_e_g, L2_bn_e_b, L2_bn_u_g, L2_bn_u_b, s2s_atom_Wih0, s2s_atom_Whh0, s2s_atom_bias0, s2s_atom_Wih1, s2s_atom_Whh1, s2s_atom_bias1, s2s_atom_Wih2, s2s_atom_Whh2, s2s_atom_bias2, s2s_bond_Wih0, s2s_bond_Whh0, s2s_bond_bias0, s2s_bond_Wih1, s2s_bond_Whh1, s2s_bond_bias1, s2s_bond_Wih2, s2s_bond_Whh2, s2s_bond_bias2, fc0_w, fc0_b, fc1_w, fc1_b, fc2_w, fc2_b):
    P = dict(locals())
    src = _pad_rows(edge_index[0].reshape(NB, 1), BP).reshape(BP // CH, CH)
    dst = _pad_rows(edge_index[1].reshape(NB, 1), BP).reshape(BP // CH, CH)
    a2g = _pad_rows(atom2graph.reshape(NA, 1), AP).reshape(AP // CH, CH)
    b2g = _pad_rows(bond2graph.reshape(NB, 1), BP).reshape(BP // CH, CH)
    na_ = _pad_rows(norm_atom, AP)
    nb_ = _pad_rows(norm_bond, BP)

    h = _mm(_pad_rows(atom_feats, AP), emb_atom_w)
    e = _mm(_pad_rows(bond_feats, BP), emb_bond_w)
    u = _mm(_pad_rows(global_feats, GP), emb_global_w)

    dims = [32, 64, 64, 32]
    ones_a = (jnp.arange(AP) < NA).astype(jnp.float32)[:, None] * jnp.ones(
        (1, 16), jnp.float32)
    ones_b = (jnp.arange(BP) < NB).astype(jnp.float32)[:, None] * jnp.ones(
        (1, 16), jnp.float32)
    ca = jnp.maximum(jnp.sum(_sc_scatter_small(a2g, ones_a, NG), axis=0)[:, :1],
                     1.0)
    cb = jnp.maximum(jnp.sum(_sc_scatter_small(b2g, ones_b, NG), axis=0)[:, :1],
                     1.0)

    for i in range(3):
        din, dout = dims[i], dims[i + 1]
        Aw = P["L%d_A_w" % i]; Ab = P["L%d_A_b" % i]
        Bw = P["L%d_B_w" % i]; Bb = P["L%d_B_b" % i]
        Cw = P["L%d_C_w" % i]; Cb = P["L%d_C_b" % i]
        Dw = P["L%d_D_w" % i]; Db = P["L%d_D_b" % i]
        Ew = P["L%d_E_w" % i]; Eb = P["L%d_E_b" % i]
        Fw = P["L%d_F_w" % i]; Fb = P["L%d_F_b" % i]

        ah = _mm(h, Aw, Ab)
        bd = _mm(h, jnp.concatenate([Bw, Dw], axis=1),
                 jnp.concatenate([Bb, jnp.zeros_like(Db)], axis=0))
        eu_tbl = _mm(u, Ew)
        gs, gd = _sc_gather(bd, [src, dst], BP)
        eu, = _sc_gather(eu_tbl, [b2g], BP)

        epre, pe = _edge_pre(e, gs, gd, eu, nb_, Cw,
                             (Cb + Db + Eb).reshape(1, dout), NB)
        st_e = _bn_scale_shift(pe, P["L%d_bn_e_g" % i], P["L%d_bn_e_b" % i], NB)
        e_new, sig, m1, m2 = _edge_post(epre, gs, gd, st_e,
                                        e if din == dout else None, NB)
        num, den = _sc_scatter_atom(src, m1, dst, m2, sig, AP)
        hpre, ph = _atom_pre(ah, num, den, na_, NA)
        st_h = _bn_scale_shift(ph, P["L%d_bn_h_g" % i], P["L%d_bn_h_b" % i], NA)
        h_new = _post_bn_relu(hpre, st_h, h if din == dout else None, NA)

        Sh = jnp.sum(_sc_scatter_small(a2g, h_new, NG), axis=0)
        Se = jnp.sum(_sc_scatter_small(b2g, e_new, NG), axis=0)
        mh = _pad_rows(Sh / ca, GP)
        me = _pad_rows(Se / cb, GP)
        u = _u_layer(u, mh, me, Fw, Fb, P["L%d_G_w" % i], P["L%d_H_w" % i],
                     P["L%d_bn_u_g" % i], P["L%d_bn_u_b" % i], din == dout)
        h, e = h_new, e_new

    s_a = _set2set(h, a2g, P, "s2s_atom", NA)
    s_b = _set2set(e, b2g, P, "s2s_bond", NB)
    x = jnp.concatenate([s_a, s_b, u[:NG]], axis=1)
    out = _mlp(_pad_rows(x, GP), fc0_w, fc0_b, fc1_w, fc1_b, fc2_w, fc2_b)
    return out[:NG]


# speedup vs baseline: 4.5038x; 1.0064x over previous
import functools

import jax
import jax.numpy as jnp
from jax import lax
from jax.experimental import pallas as pl
from jax.experimental.pallas import tpu as pltpu
from jax.experimental.pallas import tpu_sc as plsc

NA = 50000
NB = 800000
NG = 2000
BR = 2048
BP = 802816
AP = 53248
GP = 2048
NC = 2
NS = 16
CH = 128


def _mm(x, w, b=None):
    n, din = x.shape
    dout = w.shape[1]
    bias = jnp.zeros((1, dout), jnp.float32) if b is None else b.reshape(1, dout)

    def body(x_ref, w_ref, b_ref, o_ref):
        o_ref[...] = (
            jnp.dot(x_ref[...], w_ref[...], preferred_element_type=jnp.float32)
            + b_ref[...]
        )

    return pl.pallas_call(
        body,
        grid=(n // BR,),
        in_specs=[
            pl.BlockSpec((BR, din), lambda i: (i, 0)),
            pl.BlockSpec((din, dout), lambda i: (0, 0)),
            pl.BlockSpec((1, dout), lambda i: (0, 0)),
        ],
        out_specs=pl.BlockSpec((BR, dout), lambda i: (i, 0)),
        out_shape=jax.ShapeDtypeStruct((n, dout), jnp.float32),
    )(x, w, bias)


def _edge_pre(e, gs, gd, eu, nb_, w, bias, n_true):
    n, din = e.shape
    dout = w.shape[1]

    def body(e_ref, gs_ref, gd_ref, eu_ref, nb_ref, w_ref, b_ref, o_ref, p_ref):
        i = pl.program_id(0)
        t = (
            jnp.dot(e_ref[...], w_ref[...], preferred_element_type=jnp.float32)
            + b_ref[...]
        )
        t = t + gs_ref[:, dout:] + gd_ref[:, dout:] + eu_ref[...]
        t = t * nb_ref[...]
        rows = lax.broadcasted_iota(jnp.int32, (BR, 1), 0) + i * BR
        t = jnp.where(rows < n_true, t, 0.0)
        o_ref[...] = t

        @pl.when(i == 0)
        def _():
            p_ref[...] = jnp.zeros_like(p_ref)

        p_ref[0:1, :] += jnp.sum(t, axis=0, keepdims=True)
        p_ref[1:2, :] += jnp.sum(t * t, axis=0, keepdims=True)

    return pl.pallas_call(
        body,
        grid=(n // BR,),
        in_specs=[
            pl.BlockSpec((BR, din), lambda i: (i, 0)),
            pl.BlockSpec((BR, 2 * dout), lambda i: (i, 0)),
            pl.BlockSpec((BR, 2 * dout), lambda i: (i, 0)),
            pl.BlockSpec((BR, dout), lambda i: (i, 0)),
            pl.BlockSpec((BR, 1), lambda i: (i, 0)),
            pl.BlockSpec((din, dout), lambda i: (0, 0)),
            pl.BlockSpec((1, dout), lambda i: (0, 0)),
        ],
        out_specs=[
            pl.BlockSpec((BR, dout), lambda i: (i, 0)),
            pl.BlockSpec((8, dout), lambda i: (0, 0)),
        ],
        out_shape=[
            jax.ShapeDtypeStruct((n, dout), jnp.float32),
            jax.ShapeDtypeStruct((8, dout), jnp.float32),
        ],
    )(e, gs, gd, eu, nb_, w, bias)


def _edge_post(epre, gs, gd, st, e_res, n_true):
    n, dout = epre.shape
    resid = e_res is not None

    def body(*refs):
        if resid:
            ep_ref, gs_ref, gd_ref, st_ref, er_ref = refs[:5]
            outs = refs[5:]
        else:
            ep_ref, gs_ref, gd_ref, st_ref = refs[:4]
            outs = refs[4:]
        en_ref, sg_ref, m1_ref, m2_ref = outs
        i = pl.program_id(0)
        x = ep_ref[...] * st_ref[0:1, :] + st_ref[1:2, :]
        x = jnp.maximum(x, 0.0)
        if resid:
            x = x + er_ref[...]
        rows = lax.broadcasted_iota(jnp.int32, (BR, 1), 0) + i * BR
        mask = rows < n_true
        x = jnp.where(mask, x, 0.0)
        sg = jnp.where(mask, jax.nn.sigmoid(x), 0.0)
        en_ref[...] = x
        sg_ref[...] = sg
        m1_ref[...] = sg * gd_ref[:, :dout]
        m2_ref[...] = sg * gs_ref[:, :dout]

    in_specs = [
        pl.BlockSpec((BR, dout), lambda i: (i, 0)),
        pl.BlockSpec((BR, 2 * dout), lambda i: (i, 0)),
        pl.BlockSpec((BR, 2 * dout), lambda i: (i, 0)),
        pl.BlockSpec((8, dout), lambda i: (0, 0)),
    ]
    args = [epre, gs, gd, st]
    if resid:
        in_specs.append(pl.BlockSpec((BR, dout), lambda i: (i, 0)))
        args.append(e_res)
    sh = jax.ShapeDtypeStruct((n, dout), jnp.float32)
    return pl.pallas_call(
        body,
        grid=(n // BR,),
        in_specs=in_specs,
        out_specs=[pl.BlockSpec((BR, dout), lambda i: (i, 0))] * 4,
        out_shape=[sh, sh, sh, sh],
    )(*args)


def _atom_pre(ah, num, den, na_, n_true):
    n, dout = ah.shape

    def body(a_ref, nu_ref, de_ref, na_ref, o_ref, p_ref):
        i = pl.program_id(0)
        t = a_ref[...] + nu_ref[...] / (de_ref[...] + 1e-6)
        t = t * na_ref[...]
        rows = lax.broadcasted_iota(jnp.int32, (BR, 1), 0) + i * BR
        t = jnp.where(rows < n_true, t, 0.0)
        o_ref[...] = t

        @pl.when(i == 0)
        def _():
            p_ref[...] = jnp.zeros_like(p_ref)

        p_ref[0:1, :] += jnp.sum(t, axis=0, keepdims=True)
        p_ref[1:2, :] += jnp.sum(t * t, axis=0, keepdims=True)

    return pl.pallas_call(
        body,
        grid=(n // BR,),
        in_specs=[
            pl.BlockSpec((BR, dout), lambda i: (i, 0)),
            pl.BlockSpec((BR, dout), lambda i: (i, 0)),
            pl.BlockSpec((BR, dout), lambda i: (i, 0)),
            pl.BlockSpec((BR, 1), lambda i: (i, 0)),
        ],
        out_specs=[
            pl.BlockSpec((BR, dout), lambda i: (i, 0)),
            pl.BlockSpec((8, dout), lambda i: (0, 0)),
        ],
        out_shape=[
            jax.ShapeDtypeStruct((n, dout), jnp.float32),
            jax.ShapeDtypeStruct((8, dout), jnp.float32),
        ],
    )(ah, num, den, na_)


def _post_bn_relu(x, st, res, n_true):
    n, dout = x.shape
    resid = res is not None

    def body(*refs):
        if resid:
            x_ref, st_ref, r_ref, o_ref = refs
        else:
            x_ref, st_ref, o_ref = refs
        i = pl.program_id(0)
        y = jnp.maximum(x_ref[...] * st_ref[0:1, :] + st_ref[1:2, :], 0.0)
        if resid:
            y = y + r_ref[...]
        rows = lax.broadcasted_iota(jnp.int32, (BR, 1), 0) + i * BR
        o_ref[...] = jnp.where(rows < n_true, y, 0.0)

    in_specs = [
        pl.BlockSpec((BR, dout), lambda i: (i, 0)),
        pl.BlockSpec((8, dout), lambda i: (0, 0)),
    ]
    args = [x, st]
    if resid:
        in_specs.append(pl.BlockSpec((BR, dout), lambda i: (i, 0)))
        args.append(res)
    return pl.pallas_call(
        body,
        grid=(n // BR,),
        in_specs=in_specs,
        out_specs=pl.BlockSpec((BR, dout), lambda i: (i, 0)),
        out_shape=jax.ShapeDtypeStruct((n, dout), jnp.float32),
    )(*args)


def _u_layer(u_in, mh, me, Fw, fb, Gw, Hw, g, b, resid):
    din = u_in.shape[1]
    dout = Fw.shape[1]

    def body(u_ref, mh_ref, me_ref, fw_ref, fb_ref, gw_ref, hw_ref,
             gb_ref, o_ref):
        x = (
            jnp.dot(u_ref[...], fw_ref[...], preferred_element_type=jnp.float32)
            + fb_ref[...]
            + jnp.dot(mh_ref[...], gw_ref[...], preferred_element_type=jnp.float32)
            + jnp.dot(me_ref[...], hw_ref[...], preferred_element_type=jnp.float32)
        )
        rows = lax.broadcasted_iota(jnp.int32, (GP, 1), 0)
        mask = rows < NG
        x = jnp.where(mask, x, 0.0)
        mu = jnp.sum(x, axis=0, keepdims=True) / NG
        xc = x - mu
        var = jnp.sum(jnp.where(mask, xc * xc, 0.0), axis=0, keepdims=True) / NG
        y = gb_ref[0:1, :] * xc * jax.lax.rsqrt(var + 1e-5) + gb_ref[1:2, :]
        y = jnp.maximum(y, 0.0)
        if resid:
            y = y + u_ref[...]
        o_ref[...] = jnp.where(mask, y, 0.0)

    gb = jnp.concatenate(
        [g.reshape(1, dout), b.reshape(1, dout),
         jnp.zeros((6, dout), jnp.float32)], axis=0)
    return pl.pallas_call(
        body,
        grid=(1,),
        in_specs=[
            pl.BlockSpec((GP, din), lambda i: (0, 0)),
            pl.BlockSpec((GP, dout), lambda i: (0, 0)),
            pl.BlockSpec((GP, dout), lambda i: (0, 0)),
            pl.BlockSpec((din, dout), lambda i: (0, 0)),
            pl.BlockSpec((1, dout), lambda i: (0, 0)),
            pl.BlockSpec((dout, dout), lambda i: (0, 0)),
            pl.BlockSpec((dout, dout), lambda i: (0, 0)),
            pl.BlockSpec((8, dout), lambda i: (0, 0)),
        ],
        out_specs=pl.BlockSpec((GP, dout), lambda i: (0, 0)),
        out_shape=jax.ShapeDtypeStruct((GP, dout), jnp.float32),
    )(u_in, mh, me, Fw, fb.reshape(1, dout), Gw, Hw, gb)


def _lstm_iter(qs, hs, cs, w0, w1, w2, v0, v1, v2, biases, d):
    def body(qs_ref, hs_ref, cs_ref, w0_ref, w1_ref, w2_ref,
             v0_ref, v1_ref, v2_ref, b_ref, q_ref, ho_ref, co_ref):
        x = qs_ref[...]
        wr = [w0_ref, w1_ref, w2_ref]
        vr = [v0_ref, v1_ref, v2_ref]
        for l in range(3):
            h = hs_ref[:, l * d:(l + 1) * d]
            c = cs_ref[:, l * d:(l + 1) * d]
            gates = (
                jnp.dot(x, wr[l][...], preferred_element_type=jnp.float32)
                + jnp.dot(h, vr[l][...], preferred_element_type=jnp.float32)
                + b_ref[l:l + 1, :]
            )
            ii = gates[:, 0:d]
            ff = gates[:, d:2 * d]
            gg = gates[:, 2 * d:3 * d]
            oo = gates[:, 3 * d:4 * d]
            c2 = jax.nn.sigmoid(ff) * c + jax.nn.sigmoid(ii) * jnp.tanh(gg)
            h2 = jax.nn.sigmoid(oo) * jnp.tanh(c2)
            ho_ref[:, l * d:(l + 1) * d] = h2
            co_ref[:, l * d:(l + 1) * d] = c2
            x = h2
        q_ref[...] = x

    full = lambda shape: pl.BlockSpec(shape, lambda i: tuple(0 for _ in shape))
    return pl.pallas_call(
        body,
        grid=(1,),
        in_specs=[
            full((GP, 2 * d)), full((GP, 3 * d)), full((GP, 3 * d)),
            full((2 * d, 4 * d)), full((d, 4 * d)), full((d, 4 * d)),
            full((d, 4 * d)), full((d, 4 * d)), full((d, 4 * d)),
            full((8, 4 * d)),
        ],
        out_specs=[full((GP, d)), full((GP, 3 * d)), full((GP, 3 * d))],
        out_shape=[
            jax.ShapeDtypeStruct((GP, d), jnp.float32),
            jax.ShapeDtypeStruct((GP, 3 * d), jnp.float32),
            jax.ShapeDtypeStruct((GP, 3 * d), jnp.float32),
        ],
    )(qs, hs, cs, w0, w1, w2, v0, v1, v2, biases)


def _scores(feat, qseg, n_true):
    n, d = feat.shape

    def body(f_ref, q_ref, o_ref, p_ref):
        i = pl.program_id(0)
        s = jnp.sum(f_ref[...] * q_ref[...], axis=1, keepdims=True)
        rows = lax.broadcasted_iota(jnp.int32, (BR, 1), 0) + i * BR
        s = jnp.where(rows < n_true, s, -3.0e38)
        o_ref[...] = s

        @pl.when(i == 0)
        def _():
            p_ref[...] = jnp.full_like(p_ref, -3.0e38)

        m = jnp.max(s)
        p_ref[...] = jnp.maximum(p_ref[...], m)

    return pl.pallas_call(
        body,
        grid=(n // BR,),
        in_specs=[
            pl.BlockSpec((BR, d), lambda i: (i, 0)),
            pl.BlockSpec((BR, d), lambda i: (i, 0)),
        ],
        out_specs=[
            pl.BlockSpec((BR, 1), lambda i: (i, 0)),
            pl.BlockSpec((8, 128), lambda i: (0, 0)),
        ],
        out_shape=[
            jax.ShapeDtypeStruct((n, 1), jnp.float32),
            jax.ShapeDtypeStruct((8, 128), jnp.float32),
        ],
    )(feat, qseg)


def _msgs(feat, scores, gmax):
    n, d = feat.shape

    def body(f_ref, s_ref, g_ref, o_ref):
        ex = jnp.exp(s_ref[...] - g_ref[0:1, 0:1])
        o_ref[:, :d] = f_ref[...] * ex
        o_ref[:, d:d + 16] = jnp.broadcast_to(ex, (BR, 16))

    return pl.pallas_call(
        body,
        grid=(n // BR,),
        in_specs=[
            pl.BlockSpec((BR, d), lambda i: (i, 0)),
            pl.BlockSpec((BR, 1), lambda i: (i, 0)),
            pl.BlockSpec((8, 128), lambda i: (0, 0)),
        ],
        out_specs=pl.BlockSpec((BR, d + 16), lambda i: (i, 0)),
        out_shape=jax.ShapeDtypeStruct((n, d + 16), jnp.float32),
    )(feat, scores, gmax)


def _mlp(x, w0, b0, w1, b1, w2, b2):
    def body(x_ref, w0r, b0r, w1r, b1r, w2r, b2r, o_ref):
        y = jnp.maximum(
            jnp.dot(x_ref[...], w0r[...], preferred_element_type=jnp.float32)
            + b0r[...], 0.0)
        y = jnp.maximum(
            jnp.dot(y, w1r[...], preferred_element_type=jnp.float32)
            + b1r[...], 0.0)
        o_ref[...] = (
            jnp.dot(y, w2r[...], preferred_element_type=jnp.float32) + b2r[...])

    full = lambda shape: pl.BlockSpec(shape, lambda i: tuple(0 for _ in shape))
    return pl.pallas_call(
        body,
        grid=(1,),
        in_specs=[
            full((GP, 160)), full((160, 32)), full((1, 32)),
            full((32, 16)), full((1, 16)), full((16, 1)), full((1, 1)),
        ],
        out_specs=full((GP, 1)),
        out_shape=jax.ShapeDtypeStruct((GP, 1), jnp.float32),
    )(x, w0, b0.reshape(1, 32), w1, b1.reshape(1, 16), w2, b2.reshape(1, 1))


def _pick_k(nch, bytes_per_chunk):
    for k in range(16, 0, -1):
        if nch % k == 0 and k * bytes_per_chunk <= 460_000:
            return k
    return 1


def _sc_gather(table, idx2d_list, B):
    V, D = table.shape
    NW = NC * NS
    nch = B // NW // CH
    K = _pick_k(nch, CH * D * 4)
    nsi = nch // K
    nstream = len(idx2d_list)
    mesh = plsc.VectorSubcoreMesh(core_axis_name="c", subcore_axis_name="s")
    outs = tuple(jax.ShapeDtypeStruct((B, D), jnp.float32)
                 for _ in range(nstream))

    @functools.partial(
        pl.kernel, mesh=mesh,
        out_type=outs,
        compiler_params=pltpu.CompilerParams(use_tc_tiling_on_sc=False),
        scratch_types=[
            pltpu.VMEM((K, CH), jnp.int32),
            pltpu.VMEM((K * CH, D), jnp.float32),
            pltpu.SemaphoreType.DMA,
        ],
    )
    def k_fn(*refs):
        idx_refs = refs[1:1 + nstream]
        out_refs = refs[1 + nstream:1 + 2 * nstream]
        table_hbm = refs[0]
        idx_v, rows_v, sem = refs[1 + 2 * nstream:]
        wid = lax.axis_index("s") * NC + lax.axis_index("c")
        crow0 = wid * nch
        for ih, oh in zip(idx_refs, out_refs):
            def super_step(si, carry, ih=ih, oh=oh):
                crow = crow0 + si * K
                pltpu.sync_copy(ih.at[pl.ds(crow, K)], idx_v)
                hs = []
                for b in range(K):
                    hs.append(pltpu.async_copy(
                        table_hbm.at[idx_v.at[b]],
                        rows_v.at[pl.ds(b * CH, CH)], sem))
                for h in hs:
                    h.wait()
                pltpu.sync_copy(rows_v, oh.at[pl.ds(crow * CH, K * CH)])
                return carry

            lax.fori_loop(0, nsi, super_step, 0)

    res = k_fn(table, *idx2d_list)
    return tuple(res) if isinstance(res, (tuple, list)) else (res,)


def _sc_scatter_small(idx2d, msg, nrows):
    B, D = msg.shape
    nch = B // (NC * NS) // CH
    K = _pick_k(nch, CH * D * 4)
    nsi = nch // K
    rpt = nrows // NS
    z = jnp.zeros((nrows, D), jnp.float32)
    mesh = plsc.VectorSubcoreMesh(core_axis_name="c", subcore_axis_name="s")

    @functools.partial(
        pl.kernel, mesh=mesh,
        out_type=jax.ShapeDtypeStruct((NC, nrows, D), jnp.float32),
        compiler_params=pltpu.CompilerParams(use_tc_tiling_on_sc=False),
        scratch_types=[
            pltpu.VMEM((K, CH), jnp.int32),
            pltpu.VMEM((K * CH, D), jnp.float32),
            pltpu.VMEM_SHARED((nrows, D), jnp.float32),
            pltpu.SemaphoreType.DMA,
        ],
    )
    def k(idx_hbm, msg_hbm, z_hbm, out_hbm, idx_v, msg_v, shared, sem):
        c = lax.axis_index("c")
        s = lax.axis_index("s")
        wid = s * NC + c
        pltpu.sync_copy(z_hbm.at[pl.ds(s * rpt, rpt)],
                        shared.at[pl.ds(s * rpt, rpt)])
        plsc.subcore_barrier()
        crow0 = wid * nch

        def super_step(si, carry):
            crow = crow0 + si * K
            pltpu.sync_copy(idx_hbm.at[pl.ds(crow, K)], idx_v)
            pltpu.sync_copy(msg_hbm.at[pl.ds(crow * CH, K * CH)], msg_v)
            hs = []
            for b in range(K):
                hs.append(pltpu.async_copy(
                    msg_v.at[pl.ds(b * CH, CH)],
                    shared.at[idx_v.at[b]], sem, add=True))
            for h in hs:
                h.wait()
            return carry

        lax.fori_loop(0, nsi, super_step, 0)
        plsc.subcore_barrier()
        pltpu.sync_copy(shared.at[pl.ds(s * rpt, rpt)],
                        out_hbm.at[c, pl.ds(s * rpt, rpt)])

    return k(idx2d, msg, z)


def _sc_scatter_atom(idx2a, m1, idx2b, m2, sig, nrows_out):
    B, D = m1.shape
    half = D // 2
    F = half // 16
    nch = B // NS // CH
    K = _pick_k(nch, CH * 16 * 4)
    nsi = nch // K
    rpt = NA // NS
    z = jnp.zeros((NA, 16), jnp.float32)
    mesh = plsc.VectorSubcoreMesh(core_axis_name="c", subcore_axis_name="s")

    @functools.partial(
        pl.kernel, mesh=mesh,
        out_type=(jax.ShapeDtypeStruct((nrows_out, D), jnp.float32),
                  jax.ShapeDtypeStruct((nrows_out, D), jnp.float32)),
        compiler_params=pltpu.CompilerParams(use_tc_tiling_on_sc=False),
        scratch_types=[
            pltpu.VMEM((K, CH), jnp.int32),
            pltpu.VMEM((K * CH, 16), jnp.float32),
            pltpu.VMEM_SHARED((NA, 16), jnp.float32),
            pltpu.VMEM_SHARED((NA, 16), jnp.float32),
            pltpu.SemaphoreType.DMA,
        ],
    )
    def k(ia_hbm, m1_hbm, ib_hbm, m2_hbm, sg_hbm, z_hbm, num_hbm, den_hbm,
          idx_v, msg_v, sh_num, sh_den, sem):
        c = lax.axis_index("c")
        s = lax.axis_index("s")
        for f in range(F):
            fcol = c * half + f * 16
            pltpu.sync_copy(z_hbm.at[pl.ds(s * rpt, rpt)],
                            sh_num.at[pl.ds(s * rpt, rpt)])
            pltpu.sync_copy(z_hbm.at[pl.ds(s * rpt, rpt)],
                            sh_den.at[pl.ds(s * rpt, rpt)])
            plsc.subcore_barrier()
            for ih, pairs in ((ia_hbm, ((m1_hbm, sh_num), (sg_hbm, sh_den))),
                              (ib_hbm, ((m2_hbm, sh_num), (sg_hbm, sh_den)))):
                def super_step(si, carry, ih=ih, pairs=pairs):
                    crow = s * nch + si * K
                    pltpu.sync_copy(ih.at[pl.ds(crow, K)], idx_v)
                    for mh, tbl in pairs:
                        pltpu.sync_copy(
                            mh.at[pl.ds(crow * CH, K * CH), pl.ds(fcol, 16)],
                            msg_v)
                        hs = []
                        for b in range(K):
                            hs.append(pltpu.async_copy(
                                msg_v.at[pl.ds(b * CH, CH)],
                                tbl.at[idx_v.at[b]], sem, add=True))
                        for h in hs:
                            h.wait()
                    return carry

                lax.fori_loop(0, nsi, super_step, 0)
            plsc.subcore_barrier()
            pltpu.sync_copy(sh_num.at[pl.ds(s * rpt, rpt)],
                            num_hbm.at[pl.ds(s * rpt, rpt), pl.ds(fcol, 16)])
            pltpu.sync_copy(sh_den.at[pl.ds(s * rpt, rpt)],
                            den_hbm.at[pl.ds(s * rpt, rpt), pl.ds(fcol, 16)])

    return k(idx2a, m1, idx2b, m2, sig, z)


def _pad_rows(x, n):
    return jnp.pad(x, ((0, n - x.shape[0]),) + ((0, 0),) * (x.ndim - 1))


def _bn_scale_shift(part, g, b, n_true):
    mu = part[0] / n_true
    var = jnp.maximum(part[1] / n_true - mu * mu, 0.0)
    s = g / jnp.sqrt(var + 1e-5)
    t = b - mu * s
    dout = s.shape[0]
    return jnp.concatenate(
        [s.reshape(1, dout), t.reshape(1, dout),
         jnp.zeros((6, dout), jnp.float32)], axis=0)


def _set2set(feat, seg, P, prefix, n_true):
    d = feat.shape[1]
    q_star = jnp.zeros((GP, 2 * d), jnp.float32)
    hs = jnp.zeros((GP, 3 * d), jnp.float32)
    cs = jnp.zeros((GP, 3 * d), jnp.float32)
    w0 = P[prefix + "_Wih0"].T
    w1 = P[prefix + "_Wih1"].T
    w2 = P[prefix + "_Wih2"].T
    v0 = P[prefix + "_Whh0"].T
    v1 = P[prefix + "_Whh1"].T
    v2 = P[prefix + "_Whh2"].T
    biases = jnp.concatenate(
        [P[prefix + "_bias0"].reshape(1, 4 * d),
         P[prefix + "_bias1"].reshape(1, 4 * d),
         P[prefix + "_bias2"].reshape(1, 4 * d),
         jnp.zeros((5, 4 * d), jnp.float32)], axis=0)
    B = seg.shape[0] * CH
    for _ in range(5):
        q, hs, cs = _lstm_iter(q_star, hs, cs, w0, w1, w2, v0, v1, v2,
                               biases, d)
        qseg, = _sc_gather(q, [seg], B)
        scores, gmax = _scores(feat, qseg, n_true)
        m = _msgs(feat, scores, gmax)
        S = jnp.sum(_sc_scatter_small(seg, m, NG), axis=0)
        r = S[:, :d] / (S[:, d:d + 1] + 1e-12)
        q_star = jnp.concatenate([q[:NG], r], axis=1)
        q_star = _pad_rows(q_star, GP)
    return q_star[:NG]


def kernel(atom_feats, bond_feats, global_feats, edge_index, atom2graph, bond2graph, norm_atom, norm_bond, emb_atom_w, emb_bond_w, emb_global_w, L0_A_w, L0_A_b, L0_B_w, L0_B_b, L0_C_w, L0_C_b, L0_D_w, L0_D_b, L0_E_w, L0_E_b, L0_F_w, L0_F_b, L0_G_w, L0_H_w, L0_bn_h_g, L0_bn_h_b, L0_bn_e_g, L0_bn_e_b, L0_bn_u_g, L0_bn_u_b, L1_A_w, L1_A_b, L1_B_w, L1_B_b, L1_C_w, L1_C_b, L1_D_w, L1_D_b, L1_E_w, L1_E_b, L1_F_w, L1_F_b, L1_G_w, L1_H_w, L1_bn_h_g, L1_bn_h_b, L1_bn_e_g, L1_bn_e_b, L1_bn_u_g, L1_bn_u_b, L2_A_w, L2_A_b, L2_B_w, L2_B_b, L2_C_w, L2_C_b, L2_D_w, L2_D_b, L2_E_w, L2_E_b, L2_F_w, L2_F_b, L2_G_w, L2_H_w, L2_bn_h_g, L2_bn_h_b, L2_bn_e_g, L2_bn_e_b, L2_bn_u_g, L2_bn_u_b, s2s_atom_Wih0, s2s_atom_Whh0, s2s_atom_bias0, s2s_atom_Wih1, s2s_atom_Whh1, s2s_atom_bias1, s2s_atom_Wih2, s2s_atom_Whh2, s2s_atom_bias2, s2s_bond_Wih0, s2s_bond_Whh0, s2s_bond_bias0, s2s_bond_Wih1, s2s_bond_Whh1, s2s_bond_bias1, s2s_bond_Wih2, s2s_bond_Whh2, s2s_bond_bias2, fc0_w, fc0_b, fc1_w, fc1_b, fc2_w, fc2_b):
    P = dict(locals())
    src = _pad_rows(edge_index[0].reshape(NB, 1), BP).reshape(BP // CH, CH)
    dst = _pad_rows(edge_index[1].reshape(NB, 1), BP).reshape(BP // CH, CH)
    a2g = _pad_rows(atom2graph.reshape(NA, 1), AP).reshape(AP // CH, CH)
    b2g = _pad_rows(bond2graph.reshape(NB, 1), BP).reshape(BP // CH, CH)
    na_ = _pad_rows(norm_atom, AP)
    nb_ = _pad_rows(norm_bond, BP)

    h = _mm(_pad_rows(atom_feats, AP), emb_atom_w)
    e = _mm(_pad_rows(bond_feats, BP), emb_bond_w)
    u = _mm(_pad_rows(global_feats, GP), emb_global_w)

    dims = [32, 64, 64, 32]
    ones_a = (jnp.arange(AP) < NA).astype(jnp.float32)[:, None] * jnp.ones(
        (1, 16), jnp.float32)
    ones_b = (jnp.arange(BP) < NB).astype(jnp.float32)[:, None] * jnp.ones(
        (1, 16), jnp.float32)
    ca = jnp.maximum(jnp.sum(_sc_scatter_small(a2g, ones_a, NG), axis=0)[:, :1],
                     1.0)
    cb = jnp.maximum(jnp.sum(_sc_scatter_small(b2g, ones_b, NG), axis=0)[:, :1],
                     1.0)

    for i in range(3):
        din, dout = dims[i], dims[i + 1]
        Aw = P["L%d_A_w" % i]; Ab = P["L%d_A_b" % i]
        Bw = P["L%d_B_w" % i]; Bb = P["L%d_B_b" % i]
        Cw = P["L%d_C_w" % i]; Cb = P["L%d_C_b" % i]
        Dw = P["L%d_D_w" % i]; Db = P["L%d_D_b" % i]
        Ew = P["L%d_E_w" % i]; Eb = P["L%d_E_b" % i]
        Fw = P["L%d_F_w" % i]; Fb = P["L%d_F_b" % i]

        ah = _mm(h, Aw, Ab)
        bd = _mm(h, jnp.concatenate([Bw, Dw], axis=1),
                 jnp.concatenate([Bb, jnp.zeros_like(Db)], axis=0))
        eu_tbl = _mm(u, Ew)
        gs, gd = _sc_gather(bd, [src, dst], BP)
        eu, = _sc_gather(eu_tbl, [b2g], BP)

        epre, pe = _edge_pre(e, gs, gd, eu, nb_, Cw,
                             (Cb + Db + Eb).reshape(1, dout), NB)
        st_e = _bn_scale_shift(pe, P["L%d_bn_e_g" % i], P["L%d_bn_e_b" % i], NB)
        e_new, sig, m1, m2 = _edge_post(epre, gs, gd, st_e,
                                        e if din == dout else None, NB)
        num, den = _sc_scatter_atom(src, m1, dst, m2, sig, AP)
        hpre, ph = _atom_pre(ah, num, den, na_, NA)
        st_h = _bn_scale_shift(ph, P["L%d_bn_h_g" % i], P["L%d_bn_h_b" % i], NA)
        h_new = _post_bn_relu(hpre, st_h, h if din == dout else None, NA)

        Sh = jnp.sum(_sc_scatter_small(a2g, h_new, NG), axis=0)
        Se = jnp.sum(_sc_scatter_small(b2g, e_new, NG), axis=0)
        mh = _pad_rows(Sh / ca, GP)
        me = _pad_rows(Se / cb, GP)
        u = _u_layer(u, mh, me, Fw, Fb, P["L%d_G_w" % i], P["L%d_H_w" % i],
                     P["L%d_bn_u_g" % i], P["L%d_bn_u_b" % i], din == dout)
        h, e = h_new, e_new

    s_a = _set2set(h, a2g, P, "s2s_atom", NA)
    s_b = _set2set(e, b2g, P, "s2s_bond", NB)
    x = jnp.concatenate([s_a, s_b, u[:NG]], axis=1)
    out = _mlp(_pad_rows(x, GP), fc0_w, fc0_b, fc1_w, fc1_b, fc2_w, fc2_b)
    return out[:NG]
